# baseline (speedup 1.0000x reference)
import jax
import jax.numpy as jnp
from jax import lax
from jax.experimental import pallas as pl
from jax.experimental.pallas import tpu as pltpu
from jax.experimental.pallas import tpu_sc as plsc

E_ENT = 100000
B = 128
NC = 2
NS = 16
NW = NC * NS
RPW = B // NW


def _sc_body(x_hbm, zrow_hbm, out_hbm, x_v, row_v, patch_v, sem):
    wid = lax.axis_index("c") * NS + lax.axis_index("s")
    pltpu.sync_copy(x_hbm, x_v.at[pl.ds(0, B)])
    pltpu.sync_copy(zrow_hbm, row_v)
    fills = []
    for j in range(RPW):
        row = wid * RPW + j
        cp = pltpu.make_async_copy(
            row_v, out_hbm.at[pl.ds(row * E_ENT, E_ENT)], sem)
        cp.start()
        fills.append(cp)
    lane = lax.broadcasted_iota(jnp.int32, (16,), 0)
    bases = []
    for j in range(RPW):
        row = wid * RPW + j
        col = x_v[pl.ds(row, 16)][0]
        base = (col // 16) * 16
        patch_v[j] = (lane == (col - base)).astype(jnp.float32)
        bases.append((row, base))
    for cp in fills:
        cp.wait()
    patches = []
    for j, (row, base) in enumerate(bases):
        cp = pltpu.make_async_copy(
            patch_v.at[j], out_hbm.at[pl.ds(row * E_ENT + base, 16)], sem)
        cp.start()
        patches.append(cp)
    for cp in patches:
        cp.wait()


def kernel(input_x, input_r, e2triple, triple2e, r2triple, emb_table,
           W_ih, W_hh, b_ih, b_hh, W_lin, b_lin):
    x_i32 = input_x.astype(jnp.int32)
    zrow = jnp.zeros((E_ENT,), jnp.float32)
    sc = pl.kernel(
        _sc_body,
        out_type=jax.ShapeDtypeStruct((B * E_ENT,), jnp.float32),
        mesh=plsc.VectorSubcoreMesh(core_axis_name="c", subcore_axis_name="s"),
        scratch_types=[
            pltpu.VMEM((B + 16,), jnp.int32),
            pltpu.VMEM((E_ENT,), jnp.float32),
            pltpu.VMEM((RPW, 16), jnp.float32),
            pltpu.SemaphoreType.DMA,
        ],
        compiler_params=pltpu.CompilerParams(needs_layout_passes=False),
    )
    return sc(x_i32, zrow).reshape(B, E_ENT)


# speedup vs baseline: 2.2280x; 2.2280x over previous
import jax
import jax.numpy as jnp
from jax import lax
from jax.experimental import pallas as pl
from jax.experimental.pallas import tpu as pltpu

E_ENT = 100000
B = 128
NSTRIP = 16
SROWS = B // NSTRIP
NPB = 8


def _onehot_body(x_ref, out_ref, zbuf, pbuf, fsem, psem):
    zbuf[...] = jnp.zeros((SROWS, E_ENT), jnp.float32)
    fills = []
    for r in range(NSTRIP):
        cp = pltpu.make_async_copy(
            zbuf, out_ref.at[pl.ds(r * SROWS, SROWS), :], fsem.at[r])
        cp.start()
        fills.append(cp)
    for cp in fills:
        cp.wait()
    patches = []
    for i in range(B):
        xi = x_ref[i]
        cb = xi // 128
        off = xi - cb * 128
        k = i % NPB
        if i >= NPB:
            patches[i - NPB].wait()
        row = (lax.broadcasted_iota(jnp.int32, (1, 128), 1) == off).astype(
            jnp.float32)
        pbuf[pl.ds(k, 1), :] = row
        cp = pltpu.make_async_copy(
            pbuf.at[pl.ds(k, 1), :],
            out_ref.at[pl.ds(i, 1), pl.ds(cb * 128, 128)],
            psem.at[k])
        cp.start()
        patches.append(cp)
    for cp in patches[B - NPB:]:
        cp.wait()


def kernel(input_x, input_r, e2triple, triple2e, r2triple, emb_table,
           W_ih, W_hh, b_ih, b_hh, W_lin, b_lin):
    return pl.pallas_call(
        _onehot_body,
        in_specs=[pl.BlockSpec(memory_space=pltpu.SMEM)],
        out_specs=pl.BlockSpec(memory_space=pltpu.HBM),
        out_shape=jax.ShapeDtypeStruct((B, E_ENT), jnp.float32),
        scratch_shapes=[
            pltpu.VMEM((SROWS, E_ENT), jnp.float32),
            pltpu.VMEM((NPB, 128), jnp.float32),
            pltpu.SemaphoreType.DMA((NSTRIP,)),
            pltpu.SemaphoreType.DMA((NPB,)),
        ],
    )(input_x.astype(jnp.int32))


# speedup vs baseline: 8.3303x; 3.7389x over previous
import jax
import jax.numpy as jnp
from jax import lax
from jax.experimental import pallas as pl

E_ENT = 100000
B = 128
CBLK = 25000


def _onehot_t_body(x_ref, out_ref):
    j = pl.program_id(0)
    cvals = lax.broadcasted_iota(jnp.int32, (CBLK, B), 0) + j * CBLK
    out_ref[...] = (cvals == x_ref[...]).astype(jnp.float32)


def kernel(input_x, input_r, e2triple, triple2e, r2triple, emb_table,
           W_ih, W_hh, b_ih, b_hh, W_lin, b_lin):
    x2d = input_x.astype(jnp.int32).reshape(1, B)
    outT = pl.pallas_call(
        _onehot_t_body,
        grid=(E_ENT // CBLK,),
        in_specs=[pl.BlockSpec((1, B), lambda j: (0, 0))],
        out_specs=pl.BlockSpec((CBLK, B), lambda j: (j, 0)),
        out_shape=jax.ShapeDtypeStruct((E_ENT, B), jnp.float32),
    )(x2d)
    return outT.T
